# baseline (speedup 1.0000x reference)
import jax
import jax.numpy as jnp
import numpy as np
from jax import lax
from jax.experimental import pallas as pl
from jax.experimental.pallas import tpu as pltpu
from jax.experimental.pallas import tpu_sc as plsc

B, C, H, W = 8, 96, 224, 224
R = B * C
NC, NS = 2, 16
RPC = R // NC
SPL = 54
PT = (RPC - SPL) // (NS - 1)
IDXW = 64
NSB = 4
KS = 2

_out_plane = np.zeros((NC * NS, IDXW), dtype=np.int32)
for _c in range(NC):
    for _s in range(NS):
        _w = _c * NS + _s
        if _s == 0:
            _planes = _c * RPC + np.arange(SPL)
        else:
            _planes = _c * RPC + SPL + (_s - 1) * PT + np.arange(PT)
        _out_plane[_w, :len(_planes)] = _planes
_OUT_PLANE = _out_plane


def _body(x_hbm, idx_hbm, out_hbm, idx_v, bufs, gsems, ssems,
          spb, spg, sps):
    c = lax.axis_index("c")
    s = lax.axis_index("s")
    w = c * NS + s
    pltpu.sync_copy(idx_hbm.at[w], idx_v)

    lanes = lax.broadcasted_iota(jnp.int32, (16,), 0)
    vecs = [idx_v[pl.ds(16 * g, 16)] for g in range(IDXW // 16)]

    def src_of(j):
        vec, lane = vecs[j // 16], j % 16
        return lax.reduce_max(jnp.where(lanes == lane, vec, 0), (0,))

    cbase = c * RPC

    @pl.when(s == 0)
    def _spmem_path():
        base = cbase

        def gather(j):
            k = j % NSB
            return pltpu.async_copy(x_hbm.at[pl.ds(src_of(j), 1)],
                                    spb.at[pl.ds(k, 1)], spg[k])

        def wait_gather(j):
            k = j % NSB
            pltpu.make_async_copy(x_hbm.at[pl.ds(src_of(j), 1)],
                                  spb.at[pl.ds(k, 1)], spg[k]).wait()

        def scatter(j):
            k = j % NSB
            return pltpu.async_copy(spb.at[pl.ds(k, 1)],
                                    out_hbm.at[pl.ds(base + j, 1)], sps[k])

        def wait_scatter(j):
            k = j % NSB
            pltpu.make_async_copy(spb.at[pl.ds(k, 1)],
                                  out_hbm.at[pl.ds(base + j, 1)],
                                  sps[k]).wait()

        for j in range(NSB):
            gather(j)
        for j in range(SPL):
            wait_gather(j)
            scatter(j)
            if j >= KS and j - KS + NSB < SPL:
                wait_scatter(j - KS)
                gather(j - KS + NSB)
        for j in range(SPL - KS, SPL):
            wait_scatter(j)

    @pl.when(s != 0)
    def _stream_path():
        base = cbase + SPL + (s - 1) * PT

        def gather(j):
            b = j % 2
            return pltpu.async_copy(x_hbm.at[pl.ds(src_of(j), 1)], bufs[b],
                                    gsems[b])

        def wait_gather(j):
            b = j % 2
            pltpu.make_async_copy(x_hbm.at[pl.ds(src_of(j), 1)], bufs[b],
                                  gsems[b]).wait()

        def scatter(j):
            b = j % 2
            return pltpu.async_copy(bufs[b], out_hbm.at[pl.ds(base + j, 1)],
                                    ssems[b])

        def wait_scatter(j):
            b = j % 2
            pltpu.make_async_copy(bufs[b], out_hbm.at[pl.ds(base + j, 1)],
                                  ssems[b]).wait()

        gather(0)
        gather(1)
        for j in range(PT):
            wait_gather(j)
            scatter(j)
            if j + 2 < PT:
                wait_scatter(j)
                gather(j + 2)
        wait_scatter(PT - 2)
        wait_scatter(PT - 1)


@jax.jit
def kernel(x, perm):
    x3 = x.reshape(R, H, W)
    rows = jnp.arange(R, dtype=jnp.int32)
    src = (rows // C) * C + perm.astype(jnp.int32)[rows % C]
    idx_rows = src[_OUT_PLANE]

    mesh = plsc.VectorSubcoreMesh(core_axis_name="c", subcore_axis_name="s")
    out3 = pl.kernel(
        _body,
        out_type=jax.ShapeDtypeStruct((R, H, W), jnp.float32),
        mesh=mesh,
        compiler_params=pltpu.CompilerParams(use_tc_tiling_on_sc=True,
                                             needs_layout_passes=False),
        scratch_types=[
            pltpu.VMEM((IDXW,), jnp.int32),
            [pltpu.VMEM((1, H, W), jnp.float32) for _ in range(2)],
            [pltpu.SemaphoreType.DMA for _ in range(2)],
            [pltpu.SemaphoreType.DMA for _ in range(2)],
            pltpu.VMEM_SHARED((NSB, H, W), jnp.float32),
            [pltpu.SemaphoreType.DMA for _ in range(NSB)],
            [pltpu.SemaphoreType.DMA for _ in range(NSB)],
        ],
    )(x3, idx_rows)
    return out3.reshape(B, C, H, W)
